# baseline (speedup 1.0000x reference)
import functools

import jax
import jax.numpy as jnp
from jax import lax
from jax.experimental import pallas as pl
from jax.experimental.pallas import tpu as pltpu
from jax.experimental.pallas import tpu_sc as plsc

ALPHA = 200.0
B, N, M = 16, 512, 512
LANES = 16
CHUNK = 16
NCH = N // CHUNK
CVECS = M // LANES
UNROLL = 4


def _sm_body(s_hbm, nrow_hbm, ncol_hbm, out_hbm, buf, buf_t, zbuf,
             nrow_v, ncol_v, sem):
    wid = lax.axis_index("s") * 2 + lax.axis_index("c")

    pltpu.sync_copy(nrow_hbm, nrow_v)
    pltpu.sync_copy(ncol_hbm, ncol_v)

    lanes = lax.iota(jnp.int32, LANES)
    zvec = jnp.zeros((LANES,), jnp.float32)

    def _zinit(j, carry):
        zbuf[j // CVECS, pl.ds((j % CVECS) * LANES, LANES)] = zvec
        return carry

    lax.fori_loop(0, CHUNK * CVECS, _zinit, 0)

    nv = nrow_v[...]
    mv = ncol_v[...]

    def _batch(b, carry0):
        bf = jnp.full((LANES,), b, jnp.int32)
        n = nv.at[bf].get(mode="promise_in_bounds")[0]
        m = mv.at[bf].get(mode="promise_in_bounds")[0]
        ch = lax.rem(wid + 2 * b, NCH)
        r0 = ch * CHUNK
        m4 = (m // UNROLL) * UNROLL

        @pl.when(r0 < n)
        def _compute():
            pltpu.async_copy(
                s_hbm.at[b, pl.ds(r0, CHUNK), :], buf, sem).wait()
            rowv = (r0 + lanes) < n

            def _p1(g, carry):
                accs, cvec = carry
                off = g * UNROLL * LANES
                new = []
                for u in range(UNROLL):
                    x = plsc.load_gather(buf, [lanes, cvec + u])
                    buf_t[pl.ds(off + u * LANES, LANES)] = x
                    new.append(jnp.maximum(accs[u], x))
                return tuple(new), cvec + UNROLL

            def _p1_col(c, acc):
                x = plsc.load_gather(buf, [lanes, jnp.full((LANES,), c)])
                buf_t[pl.ds(c * LANES, LANES)] = x
                return jnp.maximum(acc, x)

            acc0 = jnp.full((LANES,), -3.0e38, jnp.float32)
            czero = jnp.zeros((LANES,), jnp.int32)
            mvecs, _ = lax.fori_loop(
                0, m4 // UNROLL, _p1, ((acc0,) * UNROLL, czero))
            mvec = functools.reduce(jnp.maximum, mvecs)
            rowmax = lax.fori_loop(m4, m, _p1_col, mvec)

            def _p2(g, accs):
                off = g * UNROLL * LANES
                new = []
                for u in range(UNROLL):
                    x = buf_t[pl.ds(off + u * LANES, LANES)]
                    e = jnp.exp((x - rowmax) * ALPHA)
                    buf_t[pl.ds(off + u * LANES, LANES)] = e
                    new.append(accs[u] + e)
                return tuple(new)

            def _p2_col(c, acc):
                x = buf_t[pl.ds(c * LANES, LANES)]
                e = jnp.exp((x - rowmax) * ALPHA)
                buf_t[pl.ds(c * LANES, LANES)] = e
                return acc + e

            svecs = lax.fori_loop(0, m4 // UNROLL, _p2, (zvec,) * UNROLL)
            svec = functools.reduce(jnp.add, svecs)
            denom = lax.fori_loop(m4, m, _p2_col, svec)
            scale = jnp.where(rowv, 1.0 / denom, 0.0)

            def _p3(g, cvec):
                off = g * UNROLL * LANES
                for u in range(UNROLL):
                    e = buf_t[pl.ds(off + u * LANES, LANES)]
                    plsc.store_scatter(buf, [lanes, cvec + u], e * scale)
                return cvec + UNROLL

            def _p3_col(c, carry):
                e = buf_t[pl.ds(c * LANES, LANES)]
                plsc.store_scatter(
                    buf, [lanes, jnp.full((LANES,), c)], e * scale)
                return carry

            lax.fori_loop(0, m4 // UNROLL, _p3, czero)
            lax.fori_loop(m4, m, _p3_col, 0)

            def _ztcol(c, carry):
                plsc.store_scatter(
                    buf, [lanes, jnp.full((LANES,), c)], zvec)
                return carry

            lax.fori_loop(m, M, _ztcol, 0)

            pltpu.async_copy(
                buf, out_hbm.at[b, pl.ds(r0, CHUNK), :], sem).wait()

        @pl.when(r0 >= n)
        def _zero():
            pltpu.sync_copy(zbuf, out_hbm.at[b, pl.ds(r0, CHUNK), :])

        return carry0

    lax.fori_loop(0, B, _batch, 0)


@jax.jit
def _sm_call(s, nrow_gt, ncol_gt):
    mesh = plsc.VectorSubcoreMesh(core_axis_name="c", subcore_axis_name="s")
    return pl.kernel(
        _sm_body,
        mesh=mesh,
        compiler_params=pltpu.CompilerParams(needs_layout_passes=False),
        out_type=jax.ShapeDtypeStruct((B, N, M), jnp.float32),
        scratch_types=[
            pltpu.VMEM((CHUNK, M), jnp.float32),
            pltpu.VMEM((M * CHUNK,), jnp.float32),
            pltpu.VMEM((CHUNK, M), jnp.float32),
            pltpu.VMEM((LANES,), jnp.int32),
            pltpu.VMEM((LANES,), jnp.int32),
            pltpu.SemaphoreType.DMA,
        ],
    )(s, nrow_gt, ncol_gt)


def kernel(s, nrow_gt, ncol_gt):
    return _sm_call(s, nrow_gt, ncol_gt)


# speedup vs baseline: 1.1152x; 1.1152x over previous
import functools

import jax
import jax.numpy as jnp
from jax import lax
from jax.experimental import pallas as pl
from jax.experimental.pallas import tpu as pltpu
from jax.experimental.pallas import tpu_sc as plsc

ALPHA = 200.0
LOG2E = 1.4426950408889634
BETA = ALPHA * LOG2E
B, N, M = 16, 512, 512
LANES = 16
CHUNK = 16
NCH = N // CHUNK
CVECS = M // LANES
UNROLL = 4


def _sm_body(s_hbm, nrow_hbm, ncol_hbm, out_hbm,
             buf_a, buf_b, obuf_a, obuf_b, buf_t, zbuf, nrow_v, ncol_v,
             semi_a, semi_b, semo_a, semo_b, semz):
    wid = lax.axis_index("s") * 2 + lax.axis_index("c")

    pltpu.sync_copy(nrow_hbm, nrow_v)
    pltpu.sync_copy(ncol_hbm, ncol_v)

    lanes = lax.iota(jnp.int32, LANES)
    zvec = jnp.zeros((LANES,), jnp.float32)
    czero = jnp.zeros((LANES,), jnp.int32)
    acc0 = jnp.full((LANES,), -3.0e38, jnp.float32)

    def _zinit(j, carry):
        zbuf[j // CVECS, pl.ds((j % CVECS) * LANES, LANES)] = zvec
        return carry

    lax.fori_loop(0, CHUNK * CVECS, _zinit, 0)

    nv = nrow_v[...]
    mv = ncol_v[...]

    def _nm(b):
        bf = jnp.full((LANES,), b, jnp.int32)
        n = nv.at[bf].get(mode="promise_in_bounds")[0]
        m = mv.at[bf].get(mode="promise_in_bounds")[0]
        return n, m

    def _r0(b):
        return lax.rem(wid + 2 * b, NCH) * CHUNK

    def _valid(b):
        n, _ = _nm(b)
        return _r0(b) < n

    def _in_copy(b, buf, sem):
        return pltpu.make_async_copy(
            s_hbm.at[b, pl.ds(_r0(b), CHUNK), :], buf, sem)

    def _out_copy(b, obuf, sem):
        return pltpu.make_async_copy(
            obuf, out_hbm.at[b, pl.ds(_r0(b), CHUNK), :], sem)

    @pl.when(_valid(0))
    def _pre0():
        _in_copy(0, buf_a, semi_a).start()

    @pl.when(_valid(1))
    def _pre1():
        _in_copy(1, buf_b, semi_b).start()

    def _step(b, buf, obuf, semi, semo, pend, zc):
        n, m = _nm(b)
        r0 = _r0(b)
        val = r0 < n
        m4 = (m // UNROLL) * UNROLL

        @pl.when(val)
        def _compute():
            _in_copy(b, buf, semi).wait()
            rowv = (r0 + lanes) < n

            def _p1(g, carry):
                accs, cvec = carry
                off = g * UNROLL * LANES
                new = []
                for u in range(UNROLL):
                    x = plsc.load_gather(buf, [lanes, cvec + u])
                    buf_t[pl.ds(off + u * LANES, LANES)] = x
                    new.append(jnp.maximum(accs[u], x))
                return tuple(new), cvec + UNROLL

            def _p1_col(c, acc):
                x = plsc.load_gather(buf, [lanes, jnp.full((LANES,), c)])
                buf_t[pl.ds(c * LANES, LANES)] = x
                return jnp.maximum(acc, x)

            mvecs, _ = lax.fori_loop(
                0, m4 // UNROLL, _p1, ((acc0,) * UNROLL, czero))
            mvec = functools.reduce(jnp.maximum, mvecs)
            rowmax = lax.fori_loop(m4, m, _p1_col, mvec)

            def _p2(g, accs):
                off = g * UNROLL * LANES
                new = []
                for u in range(UNROLL):
                    x = buf_t[pl.ds(off + u * LANES, LANES)]
                    e = jnp.exp((x - rowmax) * ALPHA)
                    buf_t[pl.ds(off + u * LANES, LANES)] = e
                    new.append(accs[u] + e)
                return tuple(new)

            def _p2_col(c, acc):
                x = buf_t[pl.ds(c * LANES, LANES)]
                e = jnp.exp((x - rowmax) * ALPHA)
                buf_t[pl.ds(c * LANES, LANES)] = e
                return acc + e

            svecs = lax.fori_loop(0, m4 // UNROLL, _p2, (zvec,) * UNROLL)
            svec = functools.reduce(jnp.add, svecs)
            denom = lax.fori_loop(m4, m, _p2_col, svec)
            scale = jnp.where(rowv, 1.0 / denom, 0.0)

            @pl.when(pend != 0)
            def _drain():
                _out_copy(b, obuf, semo).wait()

            def _p3(g, cvec):
                off = g * UNROLL * LANES
                for u in range(UNROLL):
                    e = buf_t[pl.ds(off + u * LANES, LANES)]
                    plsc.store_scatter(obuf, [lanes, cvec + u], e * scale)
                return cvec + UNROLL

            def _p3_col(c, carry):
                e = buf_t[pl.ds(c * LANES, LANES)]
                plsc.store_scatter(
                    obuf, [lanes, jnp.full((LANES,), c)], e * scale)
                return carry

            lax.fori_loop(0, m4 // UNROLL, _p3, czero)
            lax.fori_loop(m4, m, _p3_col, 0)

            def _ztcol(c, carry):
                plsc.store_scatter(
                    obuf, [lanes, jnp.full((LANES,), c)], zvec)
                return carry

            lax.fori_loop(m, M, _ztcol, 0)

            _out_copy(b, obuf, semo).start()

        @pl.when(jnp.logical_not(val))
        def _zero():
            pltpu.make_async_copy(
                zbuf, out_hbm.at[b, pl.ds(r0, CHUNK), :], semz).start()

        bq = jnp.minimum(b + 2, B - 1)

        @pl.when((b + 2 < B) & _valid(bq))
        def _prefetch():
            _in_copy(bq, buf, semi).start()

        pend_new = jnp.where(val, 1, pend)
        zc_new = jnp.where(val, zc, zc + 1)
        return pend_new, zc_new

    def _pair(p, carry):
        pend_a, pend_b, zc = carry
        pend_a, zc = _step(2 * p, buf_a, obuf_a, semi_a, semo_a, pend_a, zc)
        pend_b, zc = _step(2 * p + 1, buf_b, obuf_b, semi_b, semo_b,
                           pend_b, zc)
        return pend_a, pend_b, zc

    pend_a, pend_b, zc = lax.fori_loop(
        0, B // 2, _pair, (jnp.int32(0), jnp.int32(0), jnp.int32(0)))

    @pl.when(pend_a != 0)
    def _fin_a():
        _out_copy(0, obuf_a, semo_a).wait()

    @pl.when(pend_b != 0)
    def _fin_b():
        _out_copy(0, obuf_b, semo_b).wait()

    def _zdrain(i, carry):
        pltpu.make_async_copy(
            zbuf, out_hbm.at[0, pl.ds(0, CHUNK), :], semz).wait()
        return carry

    lax.fori_loop(0, zc, _zdrain, 0)


@jax.jit
def _sm_call(s, nrow_gt, ncol_gt):
    mesh = plsc.VectorSubcoreMesh(core_axis_name="c", subcore_axis_name="s")
    return pl.kernel(
        _sm_body,
        mesh=mesh,
        compiler_params=pltpu.CompilerParams(needs_layout_passes=False),
        out_type=jax.ShapeDtypeStruct((B, N, M), jnp.float32),
        scratch_types=[
            pltpu.VMEM((CHUNK, M), jnp.float32),
            pltpu.VMEM((CHUNK, M), jnp.float32),
            pltpu.VMEM((CHUNK, M), jnp.float32),
            pltpu.VMEM((CHUNK, M), jnp.float32),
            pltpu.VMEM((M * CHUNK,), jnp.float32),
            pltpu.VMEM((CHUNK, M), jnp.float32),
            pltpu.VMEM((LANES,), jnp.int32),
            pltpu.VMEM((LANES,), jnp.int32),
            pltpu.SemaphoreType.DMA,
            pltpu.SemaphoreType.DMA,
            pltpu.SemaphoreType.DMA,
            pltpu.SemaphoreType.DMA,
            pltpu.SemaphoreType.DMA,
        ],
    )(s, nrow_gt, ncol_gt)


def kernel(s, nrow_gt, ncol_gt):
    return _sm_call(s, nrow_gt, ncol_gt)


# speedup vs baseline: 1.3487x; 1.2093x over previous
import functools

import jax
import jax.numpy as jnp
from jax import lax
from jax.experimental import pallas as pl
from jax.experimental.pallas import tpu as pltpu
from jax.experimental.pallas import tpu_sc as plsc

ALPHA = 200.0
B, N, M = 16, 512, 512
LANES = 16
CHUNK = 16
NCH = N // CHUNK
CVECS = M // LANES
UNROLL = 4
NEG = -3.0e38


def _sm_body(s_hbm, nrow_hbm, ncol_hbm, out_hbm,
             buf_a, buf_b, obuf_a, obuf_b, zbuf, nrow_v, ncol_v,
             semi_a, semi_b, semo_a, semo_b, semz):
    wid = lax.axis_index("s") * 2 + lax.axis_index("c")

    pltpu.sync_copy(nrow_hbm, nrow_v)
    pltpu.sync_copy(ncol_hbm, ncol_v)

    lanes = lax.iota(jnp.int32, LANES)
    zvec = jnp.zeros((LANES,), jnp.float32)
    czero = jnp.zeros((LANES,), jnp.int32)
    acc0 = jnp.full((LANES,), NEG, jnp.float32)

    def _tree(v, op):
        for s in (8, 4, 2, 1):
            idx = jnp.bitwise_xor(lanes, s)
            v = op(v, v.at[idx].get(mode="promise_in_bounds"))
        return v

    def _zinit(j, carry):
        zbuf[j // CVECS, pl.ds((j % CVECS) * LANES, LANES)] = zvec
        return carry

    lax.fori_loop(0, CHUNK * CVECS, _zinit, 0)

    nv = nrow_v[...]
    mv = ncol_v[...]

    def _nm(b):
        bf = jnp.full((LANES,), b, jnp.int32)
        n = nv.at[bf].get(mode="promise_in_bounds")[0]
        m = mv.at[bf].get(mode="promise_in_bounds")[0]
        return n, m

    def _r0(b):
        return lax.rem(wid + 2 * b, NCH) * CHUNK

    def _valid(b):
        n, _ = _nm(b)
        return _r0(b) < n

    def _in_copy(b, buf, sem):
        return pltpu.make_async_copy(
            s_hbm.at[b, pl.ds(_r0(b), CHUNK), :], buf, sem)

    def _out_copy(b, obuf, sem):
        return pltpu.make_async_copy(
            obuf, out_hbm.at[b, pl.ds(_r0(b), CHUNK), :], sem)

    @pl.when(_valid(0))
    def _pre0():
        _in_copy(0, buf_a, semi_a).start()

    @pl.when(_valid(1))
    def _pre1():
        _in_copy(1, buf_b, semi_b).start()

    def _step(b, buf, obuf, semi, semo, pend, zc):
        n, m = _nm(b)
        r0 = _r0(b)
        val = r0 < n
        mfull = m // LANES
        rem = m - mfull * LANES
        cdiv = mfull + jnp.where(rem > 0, 1, 0)
        m4 = (mfull // UNROLL) * UNROLL
        jt = jnp.minimum(mfull, CVECS - 1)
        t0 = jt * LANES
        tcols = t0 + lanes

        @pl.when(val)
        def _compute():
            _in_copy(b, buf, semi).wait()

            @pl.when(pend != 0)
            def _drain():
                _out_copy(b, obuf, semo).wait()

            def _row(r, carry):
                def _p1(g, accs):
                    off = g * UNROLL * LANES
                    new = []
                    for u in range(UNROLL):
                        x = buf[r, pl.ds(off + u * LANES, LANES)]
                        new.append(jnp.maximum(accs[u], x))
                    return tuple(new)

                def _p1v(j, acc):
                    x = buf[r, pl.ds(j * LANES, LANES)]
                    return jnp.maximum(acc, x)

                accs = lax.fori_loop(
                    0, mfull // UNROLL, _p1, (acc0,) * UNROLL)
                acc = functools.reduce(jnp.maximum, accs)
                acc = lax.fori_loop(m4, mfull, _p1v, acc)
                xt = buf[r, pl.ds(t0, LANES)]
                acc = jnp.maximum(acc, jnp.where(tcols < m, xt, acc0))
                mrv = _tree(acc, jnp.maximum)

                def _p2(g, sums):
                    off = g * UNROLL * LANES
                    new = []
                    for u in range(UNROLL):
                        x = buf[r, pl.ds(off + u * LANES, LANES)]
                        e = jnp.exp((x - mrv) * ALPHA)
                        buf[r, pl.ds(off + u * LANES, LANES)] = e
                        new.append(sums[u] + e)
                    return tuple(new)

                def _p2v(j, sacc):
                    x = buf[r, pl.ds(j * LANES, LANES)]
                    e = jnp.exp((x - mrv) * ALPHA)
                    buf[r, pl.ds(j * LANES, LANES)] = e
                    return sacc + e

                sums = lax.fori_loop(
                    0, mfull // UNROLL, _p2, (zvec,) * UNROLL)
                sacc = functools.reduce(jnp.add, sums)
                sacc = lax.fori_loop(m4, mfull, _p2v, sacc)
                xt2 = buf[r, pl.ds(t0, LANES)]
                et = jnp.exp((xt2 - mrv) * ALPHA)
                tmask = (tcols >= mfull * LANES) & (tcols < m)
                sacc = sacc + jnp.where(tmask, et, zvec)

                @pl.when(rem > 0)
                def _p2tail():
                    buf[r, pl.ds(t0, LANES)] = jnp.where(
                        tcols < m, et, zvec)

                dv = _tree(sacc, jnp.add)
                rvv = (r0 + r + czero) < n
                srv = jnp.where(rvv, 1.0 / dv, zvec)

                def _p3(g, c):
                    off = g * UNROLL * LANES
                    for u in range(UNROLL):
                        e = buf[r, pl.ds(off + u * LANES, LANES)]
                        obuf[r, pl.ds(off + u * LANES, LANES)] = e * srv
                    return c

                def _p3v(j, c):
                    e = buf[r, pl.ds(j * LANES, LANES)]
                    obuf[r, pl.ds(j * LANES, LANES)] = e * srv
                    return c

                lax.fori_loop(0, mfull // UNROLL, _p3, 0)
                lax.fori_loop(m4, mfull, _p3v, 0)

                @pl.when(rem > 0)
                def _p3tail():
                    e = buf[r, pl.ds(t0, LANES)]
                    obuf[r, pl.ds(t0, LANES)] = e * srv

                def _pz(j, c):
                    obuf[r, pl.ds(j * LANES, LANES)] = zvec
                    return c

                lax.fori_loop(cdiv, CVECS, _pz, 0)
                return carry

            lax.fori_loop(0, CHUNK, _row, 0)
            _out_copy(b, obuf, semo).start()

        @pl.when(jnp.logical_not(val))
        def _zero():
            pltpu.make_async_copy(
                zbuf, out_hbm.at[b, pl.ds(r0, CHUNK), :], semz).start()

        bq = jnp.minimum(b + 2, B - 1)

        @pl.when((b + 2 < B) & _valid(bq))
        def _prefetch():
            _in_copy(bq, buf, semi).start()

        pend_new = jnp.where(val, 1, pend)
        zc_new = jnp.where(val, zc, zc + 1)
        return pend_new, zc_new

    def _pair(p, carry):
        pend_a, pend_b, zc = carry
        pend_a, zc = _step(2 * p, buf_a, obuf_a, semi_a, semo_a, pend_a, zc)
        pend_b, zc = _step(2 * p + 1, buf_b, obuf_b, semi_b, semo_b,
                           pend_b, zc)
        return pend_a, pend_b, zc

    pend_a, pend_b, zc = lax.fori_loop(
        0, B // 2, _pair, (jnp.int32(0), jnp.int32(0), jnp.int32(0)))

    @pl.when(pend_a != 0)
    def _fin_a():
        _out_copy(0, obuf_a, semo_a).wait()

    @pl.when(pend_b != 0)
    def _fin_b():
        _out_copy(0, obuf_b, semo_b).wait()

    def _zdrain(i, carry):
        pltpu.make_async_copy(
            zbuf, out_hbm.at[0, pl.ds(0, CHUNK), :], semz).wait()
        return carry

    lax.fori_loop(0, zc, _zdrain, 0)


@jax.jit
def _sm_call(s, nrow_gt, ncol_gt):
    mesh = plsc.VectorSubcoreMesh(core_axis_name="c", subcore_axis_name="s")
    return pl.kernel(
        _sm_body,
        mesh=mesh,
        compiler_params=pltpu.CompilerParams(needs_layout_passes=False),
        out_type=jax.ShapeDtypeStruct((B, N, M), jnp.float32),
        scratch_types=[
            pltpu.VMEM((CHUNK, M), jnp.float32),
            pltpu.VMEM((CHUNK, M), jnp.float32),
            pltpu.VMEM((CHUNK, M), jnp.float32),
            pltpu.VMEM((CHUNK, M), jnp.float32),
            pltpu.VMEM((CHUNK, M), jnp.float32),
            pltpu.VMEM((LANES,), jnp.int32),
            pltpu.VMEM((LANES,), jnp.int32),
            pltpu.SemaphoreType.DMA,
            pltpu.SemaphoreType.DMA,
            pltpu.SemaphoreType.DMA,
            pltpu.SemaphoreType.DMA,
            pltpu.SemaphoreType.DMA,
        ],
    )(s, nrow_gt, ncol_gt)


def kernel(s, nrow_gt, ncol_gt):
    return _sm_call(s, nrow_gt, ncol_gt)


# speedup vs baseline: 1.6106x; 1.1942x over previous
import functools

import jax
import jax.numpy as jnp
from jax import lax
from jax.experimental import pallas as pl
from jax.experimental.pallas import tpu as pltpu
from jax.experimental.pallas import tpu_sc as plsc

ALPHA = 200.0
B, N, M = 16, 512, 512
LANES = 16
CHUNK = 16
NCH = N // CHUNK
CVECS = M // LANES
UNROLL = 4
NEG = -3.0e38


def _sm_body(s_hbm, nrow_hbm, ncol_hbm, out_hbm,
             buf_a, buf_b, obuf_a, obuf_b, zbuf, nrow_v, ncol_v,
             semi_a, semi_b, semo_a, semo_b, semz):
    wid = lax.axis_index("s") * 2 + lax.axis_index("c")

    pltpu.sync_copy(nrow_hbm, nrow_v)
    pltpu.sync_copy(ncol_hbm, ncol_v)

    lanes = lax.iota(jnp.int32, LANES)
    zvec = jnp.zeros((LANES,), jnp.float32)
    czero = jnp.zeros((LANES,), jnp.int32)
    acc0 = jnp.full((LANES,), NEG, jnp.float32)

    def _tree(v, op):
        for s in (8, 4, 2, 1):
            idx = jnp.bitwise_xor(lanes, s)
            v = op(v, v.at[idx].get(mode="promise_in_bounds"))
        return v

    def _zinit(j, carry):
        zbuf[j // CVECS, pl.ds((j % CVECS) * LANES, LANES)] = zvec
        return carry

    lax.fori_loop(0, CHUNK * CVECS, _zinit, 0)

    nv = nrow_v[...]
    mv = ncol_v[...]

    def _nm(b):
        bf = jnp.full((LANES,), b, jnp.int32)
        n = nv.at[bf].get(mode="promise_in_bounds")[0]
        m = mv.at[bf].get(mode="promise_in_bounds")[0]
        return n, m

    def _r0(b):
        return lax.rem(wid + 2 * b, NCH) * CHUNK

    def _valid(b):
        n, _ = _nm(b)
        return _r0(b) < n

    def _in_copy(b, buf, sem):
        return pltpu.make_async_copy(
            s_hbm.at[b, pl.ds(_r0(b), CHUNK), :], buf, sem)

    def _out_copy(b, obuf, sem):
        return pltpu.make_async_copy(
            obuf, out_hbm.at[b, pl.ds(_r0(b), CHUNK), :], sem)

    @pl.when(_valid(0))
    def _pre0():
        _in_copy(0, buf_a, semi_a).start()

    @pl.when(_valid(1))
    def _pre1():
        _in_copy(1, buf_b, semi_b).start()

    def _step(b, buf, obuf, semi, semo, pend, zc):
        n, m = _nm(b)
        r0 = _r0(b)
        val = r0 < n
        mfull = m // LANES
        rem = m - mfull * LANES
        cdiv = mfull + jnp.where(rem > 0, 1, 0)
        m4 = (mfull // UNROLL) * UNROLL
        jt = jnp.minimum(mfull, CVECS - 1)
        t0 = jt * LANES
        tcols = t0 + lanes

        @pl.when(val)
        def _compute():
            _in_copy(b, buf, semi).wait()

            @pl.when(pend != 0)
            def _drain():
                _out_copy(b, obuf, semo).wait()

            def _row(r, carry):
                def _p1(g, accs):
                    off = g * UNROLL * LANES
                    new = []
                    for u in range(UNROLL):
                        x = buf[r, pl.ds(off + u * LANES, LANES)]
                        new.append(jnp.maximum(accs[u], x))
                    return tuple(new)

                def _p1v(j, acc):
                    x = buf[r, pl.ds(j * LANES, LANES)]
                    return jnp.maximum(acc, x)

                accs = lax.fori_loop(
                    0, mfull // UNROLL, _p1, (acc0,) * UNROLL)
                acc = functools.reduce(jnp.maximum, accs)
                acc = lax.fori_loop(m4, mfull, _p1v, acc)
                xt = buf[r, pl.ds(t0, LANES)]
                acc = jnp.maximum(acc, jnp.where(tcols < m, xt, acc0))
                mrv = _tree(acc, jnp.maximum)

                def _p2(g, sums):
                    off = g * UNROLL * LANES
                    new = []
                    for u in range(UNROLL):
                        x = buf[r, pl.ds(off + u * LANES, LANES)]
                        new.append(sums[u] + jnp.exp((x - mrv) * ALPHA))
                    return tuple(new)

                def _p2v(j, sacc):
                    x = buf[r, pl.ds(j * LANES, LANES)]
                    return sacc + jnp.exp((x - mrv) * ALPHA)

                sums = lax.fori_loop(
                    0, mfull // UNROLL, _p2, (zvec,) * UNROLL)
                sacc = functools.reduce(jnp.add, sums)
                sacc = lax.fori_loop(m4, mfull, _p2v, sacc)
                xt2 = buf[r, pl.ds(t0, LANES)]
                et = jnp.exp((xt2 - mrv) * ALPHA)
                tmask = (tcols >= mfull * LANES) & (tcols < m)
                sacc = sacc + jnp.where(tmask, et, zvec)

                dv = _tree(sacc, jnp.add)
                rvv = (r0 + r + czero) < n
                srv = jnp.where(rvv, 1.0 / dv, zvec)

                def _p3(g, c):
                    off = g * UNROLL * LANES
                    for u in range(UNROLL):
                        x = buf[r, pl.ds(off + u * LANES, LANES)]
                        e = jnp.exp((x - mrv) * ALPHA)
                        obuf[r, pl.ds(off + u * LANES, LANES)] = e * srv
                    return c

                def _p3v(j, c):
                    x = buf[r, pl.ds(j * LANES, LANES)]
                    e = jnp.exp((x - mrv) * ALPHA)
                    obuf[r, pl.ds(j * LANES, LANES)] = e * srv
                    return c

                lax.fori_loop(0, mfull // UNROLL, _p3, 0)
                lax.fori_loop(m4, mfull, _p3v, 0)

                @pl.when(rem > 0)
                def _p3tail():
                    obuf[r, pl.ds(t0, LANES)] = jnp.where(
                        tcols < m, et * srv, zvec)

                def _pz(j, c):
                    obuf[r, pl.ds(j * LANES, LANES)] = zvec
                    return c

                lax.fori_loop(cdiv, CVECS, _pz, 0)
                return carry

            lax.fori_loop(0, CHUNK, _row, 0)
            _out_copy(b, obuf, semo).start()

        @pl.when(jnp.logical_not(val))
        def _zero():
            pltpu.make_async_copy(
                zbuf, out_hbm.at[b, pl.ds(r0, CHUNK), :], semz).start()

        bq = jnp.minimum(b + 2, B - 1)

        @pl.when((b + 2 < B) & _valid(bq))
        def _prefetch():
            _in_copy(bq, buf, semi).start()

        pend_new = jnp.where(val, 1, pend)
        zc_new = jnp.where(val, zc, zc + 1)
        return pend_new, zc_new

    def _pair(p, carry):
        pend_a, pend_b, zc = carry
        pend_a, zc = _step(2 * p, buf_a, obuf_a, semi_a, semo_a, pend_a, zc)
        pend_b, zc = _step(2 * p + 1, buf_b, obuf_b, semi_b, semo_b,
                           pend_b, zc)
        return pend_a, pend_b, zc

    pend_a, pend_b, zc = lax.fori_loop(
        0, B // 2, _pair, (jnp.int32(0), jnp.int32(0), jnp.int32(0)))

    @pl.when(pend_a != 0)
    def _fin_a():
        _out_copy(0, obuf_a, semo_a).wait()

    @pl.when(pend_b != 0)
    def _fin_b():
        _out_copy(0, obuf_b, semo_b).wait()

    def _zdrain(i, carry):
        pltpu.make_async_copy(
            zbuf, out_hbm.at[0, pl.ds(0, CHUNK), :], semz).wait()
        return carry

    lax.fori_loop(0, zc, _zdrain, 0)


@jax.jit
def _sm_call(s, nrow_gt, ncol_gt):
    mesh = plsc.VectorSubcoreMesh(core_axis_name="c", subcore_axis_name="s")
    return pl.kernel(
        _sm_body,
        mesh=mesh,
        compiler_params=pltpu.CompilerParams(needs_layout_passes=False),
        out_type=jax.ShapeDtypeStruct((B, N, M), jnp.float32),
        scratch_types=[
            pltpu.VMEM((CHUNK, M), jnp.float32),
            pltpu.VMEM((CHUNK, M), jnp.float32),
            pltpu.VMEM((CHUNK, M), jnp.float32),
            pltpu.VMEM((CHUNK, M), jnp.float32),
            pltpu.VMEM((CHUNK, M), jnp.float32),
            pltpu.VMEM((LANES,), jnp.int32),
            pltpu.VMEM((LANES,), jnp.int32),
            pltpu.SemaphoreType.DMA,
            pltpu.SemaphoreType.DMA,
            pltpu.SemaphoreType.DMA,
            pltpu.SemaphoreType.DMA,
            pltpu.SemaphoreType.DMA,
        ],
    )(s, nrow_gt, ncol_gt)


def kernel(s, nrow_gt, ncol_gt):
    return _sm_call(s, nrow_gt, ncol_gt)
